# baseline (speedup 1.0000x reference)
import functools

import jax
import jax.numpy as jnp
from jax import lax
from jax.experimental import pallas as pl
from jax.experimental.pallas import tpu as pltpu
from jax.experimental.pallas import tpu_sc as plsc

_T, _D, _F, _E = 1024, 768, 1024, 8
_A = 2 * _T
_LIMIT = 7.0
_ALPHA = 1.702
_BR = 128
_NT = (_A + _E * (_BR - 1) + _BR - 1) // _BR
_NROWS = _NT * _BR
_NW = 32
_TPW = _T // _NW


def _router_meta_body(x_ref, w_ref, b_ref, pos_ref, wgt_ref, texp_ref, nact_ref):
    logits = lax.dot_general(w_ref[...], x_ref[...], (((0,), (1,)), ((), ())),
                             preferred_element_type=jnp.float32)
    logits = logits + b_ref[...]
    eidx = lax.broadcasted_iota(jnp.int32, (_E, _T), 0)
    m1 = jnp.max(logits, axis=0, keepdims=True)
    a1 = jnp.min(jnp.where(logits == m1, eidx, _E), axis=0, keepdims=True)
    l2 = jnp.where(eidx == a1, -jnp.inf, logits)
    m2 = jnp.max(l2, axis=0, keepdims=True)
    a2 = jnp.min(jnp.where(l2 == m2, eidx, _E), axis=0, keepdims=True)
    w1 = jax.nn.sigmoid(m1 - m2)

    oh = jnp.concatenate([eidx == a1, eidx == a2], axis=1).astype(jnp.float32)
    lidx = lax.broadcasted_iota(jnp.int32, (_E, _A), 1)
    acc = oh
    k = 1
    while k < _A:
        acc = acc + jnp.where(lidx >= k, pltpu.roll(acc, k, 1), 0.0)
        k *= 2
    ranks = acc - oh
    counts = jnp.sum(oh, axis=1, keepdims=True)
    pc = ((counts.astype(jnp.int32) + (_BR - 1)) // _BR) * _BR
    er = lax.broadcasted_iota(jnp.int32, (_E, 1), 0)
    poff = jnp.zeros((_E, 1), jnp.int32)
    for k in range(1, _E):
        poff = poff + jnp.where(er >= k, pltpu.roll(pc, k, 0), 0)
    poff_f = poff.astype(jnp.float32)
    rank_i = jnp.sum(oh * ranks, axis=0, keepdims=True)
    poff_i = jnp.sum(oh * poff_f, axis=0, keepdims=True)
    pos_ref[...] = (rank_i + poff_i).astype(jnp.int32)
    wgt_ref[...] = jnp.concatenate([w1, 1.0 - w1], axis=1)
    tb = lax.broadcasted_iota(jnp.int32, (1, _NT), 1) * _BR
    texp_ref[...] = jnp.sum((poff <= tb).astype(jnp.int32), axis=0, keepdims=True) - 1
    nact_ref[...] = jnp.sum(pc, axis=0, keepdims=True) // _BR


def _grouped_mlp_body(texp_ref, nact_ref, xs_ref, w1_ref, b1_ref, w2_ref, b2_ref, rows_ref):
    j = pl.program_id(0)

    @pl.when(j < nact_ref[0])
    def _():
        x = xs_ref[...]
        gu = jnp.dot(x, w1_ref[0], preferred_element_type=jnp.float32) + b1_ref[0]
        gate = jnp.minimum(gu[:, :_F], _LIMIT)
        up = jnp.clip(gu[:, _F:], -_LIMIT, _LIMIT)
        act = (up + 1.0) * (gate * jax.nn.sigmoid(_ALPHA * gate))
        rows_ref[...] = (jnp.dot(act, w2_ref[0],
                                 preferred_element_type=jnp.float32) + b2_ref[0])


@functools.lru_cache(maxsize=None)
def _sc_kernels():
    mesh = plsc.VectorSubcoreMesh(core_axis_name="c", subcore_axis_name="s")

    @functools.partial(
        pl.kernel, mesh=mesh,
        out_type=jax.ShapeDtypeStruct((_NROWS, _D), jnp.float32),
        scratch_types=[
            pltpu.VMEM((_TPW,), jnp.int32),
            pltpu.VMEM((_TPW,), jnp.int32),
            pltpu.VMEM((_TPW, _D), jnp.float32),
            pltpu.SemaphoreType.DMA,
        ],
    )
    def _dispatch(x_hbm, pos_hbm, xs_hbm, idx0_v, idx1_v, rows_v, sem):
        wid = lax.axis_index("s") * 2 + lax.axis_index("c")
        base = wid * _TPW
        pltpu.sync_copy(x_hbm.at[pl.ds(base, _TPW)], rows_v)
        pltpu.sync_copy(pos_hbm.at[pl.ds(base, _TPW)], idx0_v)
        pltpu.sync_copy(pos_hbm.at[pl.ds(_T + base, _TPW)], idx1_v)
        pltpu.async_copy(rows_v, xs_hbm.at[idx0_v], sem).wait()
        pltpu.async_copy(rows_v, xs_hbm.at[idx1_v], sem).wait()

    @functools.partial(
        pl.kernel, mesh=mesh,
        out_type=jax.ShapeDtypeStruct((_T, _D), jnp.float32),
        scratch_types=[
            pltpu.VMEM((_TPW,), jnp.int32),
            pltpu.VMEM((_TPW,), jnp.int32),
            pltpu.VMEM((_TPW,), jnp.float32),
            pltpu.VMEM((_TPW,), jnp.float32),
            pltpu.VMEM((_TPW, _D), jnp.float32),
            pltpu.VMEM((_TPW, _D), jnp.float32),
            pltpu.SemaphoreType.DMA,
        ],
    )
    def _combine(rows_hbm, pos_hbm, wgt_hbm, out_hbm, idx0_v, idx1_v, w0_v, w1_v, r0_v, r1_v, sem):
        wid = lax.axis_index("s") * 2 + lax.axis_index("c")
        base = wid * _TPW
        pltpu.sync_copy(pos_hbm.at[pl.ds(base, _TPW)], idx0_v)
        pltpu.sync_copy(pos_hbm.at[pl.ds(_T + base, _TPW)], idx1_v)
        pltpu.sync_copy(wgt_hbm.at[pl.ds(base, _TPW)], w0_v)
        pltpu.sync_copy(wgt_hbm.at[pl.ds(_T + base, _TPW)], w1_v)
        c0 = pltpu.async_copy(rows_hbm.at[idx0_v], r0_v, sem)
        c1 = pltpu.async_copy(rows_hbm.at[idx1_v], r1_v, sem)
        c0.wait()
        c1.wait()

        lane = lax.broadcasted_iota(jnp.int32, (16,), 0)
        for g in range(_TPW // 16):
            w0c = w0_v[pl.ds(g * 16, 16)]
            w1c = w1_v[pl.ds(g * 16, 16)]

            def row_body(rr, carry, w0c=w0c, w1c=w1c, g=g):
                idxv = lane * 0 + rr
                w0b = w0c.at[idxv].get(mode="promise_in_bounds")
                w1b = w1c.at[idxv].get(mode="promise_in_bounds")
                r = g * 16 + rr
                for c in range(_D // 16):
                    sl = pl.ds(c * 16, 16)
                    r0_v[r, sl] = w0b * r0_v[r, sl] + w1b * r1_v[r, sl]
                return carry

            lax.fori_loop(0, 16, row_body, 0)
        pltpu.sync_copy(r0_v, out_hbm.at[pl.ds(base, _TPW)])

    return _dispatch, _combine


@jax.jit
def kernel(x_TD, kernel_DE, bias_E, mlp1_weight_EDF2, mlp1_bias_EF2, mlp2_weight_EFD, mlp2_bias_ED):
    x = x_TD.astype(jnp.float32)
    pos2, wgt2, texp2, nact2 = pl.pallas_call(
        _router_meta_body,
        out_shape=(
            jax.ShapeDtypeStruct((1, _A), jnp.int32),
            jax.ShapeDtypeStruct((1, _A), jnp.float32),
            jax.ShapeDtypeStruct((1, _NT), jnp.int32),
            jax.ShapeDtypeStruct((1, 1), jnp.int32),
        ),
    )(x, kernel_DE, bias_E.reshape(_E, 1))
    pos = pos2.reshape(_A)
    wgt = wgt2.reshape(_A)
    texp = texp2.reshape(_NT)
    nact = nact2.reshape(1)

    dispatch_k, combine_k = _sc_kernels()
    xs = dispatch_k(x, pos)

    rows = pl.pallas_call(
        _grouped_mlp_body,
        grid_spec=pltpu.PrefetchScalarGridSpec(
            num_scalar_prefetch=2,
            grid=(_NT,),
            in_specs=[
                pl.BlockSpec((_BR, _D), lambda j, texp, nact: (j, 0)),
                pl.BlockSpec((1, _D, 2 * _F), lambda j, texp, nact: (texp[j], 0, 0)),
                pl.BlockSpec((1, 1, 2 * _F), lambda j, texp, nact: (texp[j], 0, 0)),
                pl.BlockSpec((1, _F, _D), lambda j, texp, nact: (texp[j], 0, 0)),
                pl.BlockSpec((1, 1, _D), lambda j, texp, nact: (texp[j], 0, 0)),
            ],
            out_specs=pl.BlockSpec((_BR, _D), lambda j, texp, nact: (j, 0)),
        ),
        out_shape=jax.ShapeDtypeStruct((_NROWS, _D), jnp.float32),
        compiler_params=pltpu.CompilerParams(dimension_semantics=("arbitrary",)),
    )(texp, nact, xs, mlp1_weight_EDF2, mlp1_bias_EF2.reshape(_E, 1, 2 * _F),
      mlp2_weight_EFD, mlp2_bias_ED.reshape(_E, 1, _D))

    return combine_k(rows, pos, wgt)


# speedup vs baseline: 1.3997x; 1.3997x over previous
import functools

import jax
import jax.numpy as jnp
from jax.experimental import pallas as pl
from jax.experimental.pallas import tpu as pltpu

_T, _D, _F, _E = 1024, 768, 1024, 8
_LIMIT = 7.0
_ALPHA = 1.702


def _router_body(x_ref, w_ref, b_ref, comb_ref):
    x = x_ref[...]
    logits = jnp.dot(x, w_ref[...], preferred_element_type=jnp.float32) + b_ref[...]
    idx = jax.lax.broadcasted_iota(jnp.int32, (_T, _E), 1)
    m1 = jnp.max(logits, axis=1, keepdims=True)
    a1 = jnp.min(jnp.where(logits == m1, idx, _E), axis=1, keepdims=True)
    l2 = jnp.where(idx == a1, -jnp.inf, logits)
    m2 = jnp.max(l2, axis=1, keepdims=True)
    a2 = jnp.min(jnp.where(l2 == m2, idx, _E), axis=1, keepdims=True)
    w1 = jax.nn.sigmoid(m1 - m2)
    comb_ref[...] = jnp.where(idx == a1, w1, 0.0) + jnp.where(idx == a2, 1.0 - w1, 0.0)


def _expert_body(comb_ref, x_ref, w1_ref, b1_ref, w2_ref, b2_ref, out_ref):
    e = pl.program_id(0)
    x = x_ref[...]
    gu = jnp.dot(x, w1_ref[0], preferred_element_type=jnp.float32) + b1_ref[0]
    gate = jnp.minimum(gu[:, :_F], _LIMIT)
    up = jnp.clip(gu[:, _F:], -_LIMIT, _LIMIT)
    act = (up + 1.0) * (gate * jax.nn.sigmoid(_ALPHA * gate))
    out = jnp.dot(act, w2_ref[0], preferred_element_type=jnp.float32) + b2_ref[0]
    idx = jax.lax.broadcasted_iota(jnp.int32, (_T, _E), 1)
    c = jnp.sum(jnp.where(idx == e, comb_ref[...], 0.0), axis=1, keepdims=True)
    contrib = c * out

    @pl.when(e == 0)
    def _():
        out_ref[...] = contrib

    @pl.when(e > 0)
    def _():
        out_ref[...] += contrib


@jax.jit
def kernel(x_TD, kernel_DE, bias_E, mlp1_weight_EDF2, mlp1_bias_EF2, mlp2_weight_EFD, mlp2_bias_ED):
    x = x_TD.astype(jnp.float32)
    comb = pl.pallas_call(
        _router_body,
        out_shape=jax.ShapeDtypeStruct((_T, _E), jnp.float32),
    )(x, kernel_DE, bias_E.reshape(1, _E))

    out = pl.pallas_call(
        _expert_body,
        grid=(_E,),
        in_specs=[
            pl.BlockSpec((_T, _E), lambda e: (0, 0)),
            pl.BlockSpec((_T, _D), lambda e: (0, 0)),
            pl.BlockSpec((1, _D, 2 * _F), lambda e: (e, 0, 0)),
            pl.BlockSpec((1, 1, 2 * _F), lambda e: (e, 0, 0)),
            pl.BlockSpec((1, _F, _D), lambda e: (e, 0, 0)),
            pl.BlockSpec((1, 1, _D), lambda e: (e, 0, 0)),
        ],
        out_specs=pl.BlockSpec((_T, _D), lambda e: (0, 0)),
        out_shape=jax.ShapeDtypeStruct((_T, _D), jnp.float32),
        compiler_params=pltpu.CompilerParams(
            dimension_semantics=("arbitrary",),
        ),
    )(comb, x, mlp1_weight_EDF2, mlp1_bias_EF2.reshape(_E, 1, 2 * _F),
      mlp2_weight_EFD, mlp2_bias_ED.reshape(_E, 1, _D))
    return out.astype(jnp.float32)
